# baseline (speedup 1.0000x reference)
import functools

import jax
import jax.numpy as jnp
from jax import lax
from jax.experimental import pallas as pl
from jax.experimental.pallas import tpu as pltpu
from jax.experimental.pallas import tpu_sc as plsc

B = 4
S = 2048
H = 128
TAB = 500000
NSLOT = 7

_CMOD = (1, 256, 65536, 277216, 467296, 127776, 210656, 427936)
_WRAP = 448384

_NC = 2
_NS = 16
_NW = _NC * _NS

_SHALF = S // 2
_CHUNK = 128
_CPBH = _SHALF // _CHUNK
_NBUF = 3


def _make_sc_body(half):
    t_half = half * _SHALF

    def body(x_hbm, byte_hbm, t3, t4, t5, t6, t7, t8, out_hbm,
             xv, idxv0, idxv1, idxv2, rowsv0, rowsv1, rowsv2,
             gsem0, gsem1, gsem2, wsem0, wsem1, wsem2):
        wid = lax.axis_index("s") * _NC + lax.axis_index("c")
        pltpu.sync_copy(x_hbm, xv.at[pl.ds(0, B * S)])
        xv[pl.ds(B * S, 16)] = jnp.zeros((16,), jnp.int32)

        tables = (t3, t4, t5, t6, t7, t8, byte_hbm)
        c500k = jnp.full((16,), 500000, jnp.int32)
        zeros16 = jnp.zeros((16,), jnp.int32)
        wrap16 = jnp.full((16,), _WRAP, jnp.int32)

        idxv = (idxv0, idxv1, idxv2)
        rowsv = (rowsv0, rowsv1, rowsv2)
        gsem = (gsem0, gsem1, gsem2)
        wsem = (wsem0, wsem1, wsem2)

        b = wid // _CPBH
        t0l = (wid - b * _CPBH) * _CHUNK
        base = b * S + t_half + t0l

        def compute_idx(slot, n, p):
            for g in range(_CHUNK // 16):
                off = base + g * 16
                if slot == 6:
                    h = xv[pl.ds(off, 16)]
                else:
                    acc = xv[pl.ds(off, 16)]
                    for i in range(1, n):
                        acc = acc + xv[pl.ds(off + i, 16)] * _CMOD[i]
                    if n == 8:
                        x7 = xv[pl.ds(off + 7, 16)]
                        acc = acc + jnp.where(x7 >= 128, wrap16, zeros16)
                    h = jnp.maximum(lax.rem(acc, c500k), zeros16)
                idxv[p][pl.ds(g * 16, 16)] = h
            return (slot * B + b) * _SHALF + t0l

        pend_g = [None] * _NBUF
        pend_w = [None] * _NBUF
        out_base_of = [None] * _NBUF
        for i in range(NSLOT):
            p = i % _NBUF
            if pend_w[p] is not None:
                pend_w[p].wait()
                pend_w[p] = None
            out_base_of[p] = compute_idx(i, i + 3, p)
            pend_g[p] = pltpu.async_copy(tables[i].at[idxv[p]], rowsv[p],
                                         gsem[p])
            q = (i - 1) % _NBUF
            if i >= 1 and pend_g[q] is not None:
                pend_g[q].wait()
                pend_g[q] = None
                pend_w[q] = pltpu.async_copy(
                    rowsv[q], out_hbm.at[pl.ds(out_base_of[q], _CHUNK)],
                    wsem[q])
        p = (NSLOT - 1) % _NBUF
        pend_g[p].wait()
        pend_w[p] = pltpu.async_copy(
            rowsv[p], out_hbm.at[pl.ds(out_base_of[p], _CHUNK)], wsem[p])
        for q in range(_NBUF):
            if pend_w[q] is not None:
                pend_w[q].wait()

    return body


@functools.cache
def _build_sc_gather(half):
    mesh = plsc.VectorSubcoreMesh(core_axis_name="c", subcore_axis_name="s")
    return pl.kernel(
        _make_sc_body(half),
        out_type=jax.ShapeDtypeStruct((NSLOT * B * _SHALF, H), jnp.float32),
        mesh=mesh,
        scratch_types=(
            [pltpu.VMEM((B * S + 16,), jnp.int32)]
            + [pltpu.VMEM((_CHUNK,), jnp.int32) for _ in range(_NBUF)]
            + [pltpu.VMEM((_CHUNK, H), jnp.float32) for _ in range(_NBUF)]
            + [pltpu.SemaphoreType.DMA for _ in range(2 * _NBUF)]
        ),
    )


_TBLK = 512


def _make_tc_body(half):
    t_half = half * _SHALF

    def body(f_ref, w_ref, b_ref, o_ref):
        tb = pl.program_id(1)
        acc = f_ref[6, 0] + b_ref[0][None, :]
        row = (lax.broadcasted_iota(jnp.int32, (_TBLK, H), 0)
               + tb * _TBLK + t_half)
        for k in range(6):
            n = k + 3
            f = f_ref[k, 0]
            f = jnp.where(row <= S - n, f, 0.0)
            wk = w_ref[:, k * H:(k + 1) * H]
            acc = acc + lax.dot_general(
                f, wk, (((1,), (1,)), ((), ())),
                preferred_element_type=jnp.float32)
        o_ref[0] = acc

    return body


def _build_tc_project(half):
    return pl.pallas_call(
        _make_tc_body(half),
        grid=(B, _SHALF // _TBLK),
        in_specs=[
            pl.BlockSpec((NSLOT, 1, _TBLK, H),
                         lambda b, t: (jnp.int32(0), b, t, jnp.int32(0))),
            pl.BlockSpec((H, 6 * H),
                         lambda b, t: (jnp.int32(0), jnp.int32(0))),
            pl.BlockSpec((1, H), lambda b, t: (jnp.int32(0), jnp.int32(0))),
        ],
        out_specs=pl.BlockSpec((1, _TBLK, H),
                               lambda b, t: (b, t, jnp.int32(0))),
        out_shape=jax.ShapeDtypeStruct((B, _SHALF, H), jnp.float32),
    )


_tc_project_0 = _build_tc_project(0)
_tc_project_1 = _build_tc_project(1)


def kernel(x, byte_table, ngram_3, ngram_4, ngram_5, ngram_6, ngram_7,
           ngram_8, W, b):
    x32 = x.astype(jnp.int32).reshape(B * S)
    tabs = (byte_table, ngram_3, ngram_4, ngram_5, ngram_6, ngram_7, ngram_8)
    b2 = b.reshape(1, H)
    feats0 = _build_sc_gather(0)(x32, *tabs)
    feats1 = _build_sc_gather(1)(x32, *tabs)
    out0 = _tc_project_0(feats0.reshape(NSLOT, B, _SHALF, H), W, b2)
    out1 = _tc_project_1(feats1.reshape(NSLOT, B, _SHALF, H), W, b2)
    return jnp.concatenate([out0, out1], axis=1)


# speedup vs baseline: 1.0207x; 1.0207x over previous
import functools

import jax
import jax.numpy as jnp
from jax import lax
from jax.experimental import pallas as pl
from jax.experimental.pallas import tpu as pltpu
from jax.experimental.pallas import tpu_sc as plsc

B = 4
S = 2048
H = 128
TAB = 500000
NSLOT = 7

_CMOD = (1, 256, 65536, 277216, 467296, 127776, 210656, 427936)
_WRAP = 448384

_NC = 2
_NS = 16
_NW = _NC * _NS

_CHUNK = 128
_CPB = S // _CHUNK
_TPW = (B * _CPB) // _NW
_NTASK = NSLOT * _TPW
_NBUF = 6


def _sc_gather_body(x_hbm, byte_hbm, t3, t4, t5, t6, t7, t8, out_hbm,
                    xv, idxv, rowsv0, rowsv1, rowsv2, rowsv3, rowsv4, rowsv5,
                    gsem0, gsem1, gsem2, gsem3, gsem4, gsem5,
                    wsem0, wsem1, wsem2, wsem3, wsem4, wsem5):
    wid = lax.axis_index("s") * _NC + lax.axis_index("c")
    pltpu.sync_copy(x_hbm, xv.at[pl.ds(0, B * S)])
    xv[pl.ds(B * S, 16)] = jnp.zeros((16,), jnp.int32)

    tables = (t3, t4, t5, t6, t7, t8, byte_hbm)
    c500k = jnp.full((16,), 500000, jnp.int32)
    zeros16 = jnp.zeros((16,), jnp.int32)
    wrap16 = jnp.full((16,), _WRAP, jnp.int32)

    rowsv = (rowsv0, rowsv1, rowsv2, rowsv3, rowsv4, rowsv5)
    gsem = (gsem0, gsem1, gsem2, gsem3, gsem4, gsem5)
    wsem = (wsem0, wsem1, wsem2, wsem3, wsem4, wsem5)

    tasks = [(slot, j) for slot in range(NSLOT) for j in range(_TPW)]

    out_base_of = []
    for i, (slot, j) in enumerate(tasks):
        n = slot + 3
        task = wid * _TPW + j
        b = task // _CPB
        t0 = (task - b * _CPB) * _CHUNK
        base = b * S + t0
        for g in range(_CHUNK // 16):
            off = base + g * 16
            if slot == 6:
                h = xv[pl.ds(off, 16)]
            else:
                acc = xv[pl.ds(off, 16)]
                for ii in range(1, n):
                    acc = acc + xv[pl.ds(off + ii, 16)] * _CMOD[ii]
                if n == 8:
                    x7 = xv[pl.ds(off + 7, 16)]
                    acc = acc + jnp.where(x7 >= 128, wrap16, zeros16)
                h = jnp.maximum(lax.rem(acc, c500k), zeros16)
            idxv[i, pl.ds(g * 16, 16)] = h
        out_base_of.append((slot * B + b) * S + t0)

    pend_g = [None] * _NTASK
    pend_w = [None] * _NBUF
    for i, (slot, j) in enumerate(tasks):
        p = i % _NBUF
        if pend_w[p] is not None:
            pend_w[p].wait()
            pend_w[p] = None
        pend_g[i] = pltpu.async_copy(
            tables[slot].at[idxv.at[jnp.int32(i)]], rowsv[p], gsem[p])
        m = i - (_NBUF - 1)
        if m >= 0:
            q = m % _NBUF
            pend_g[m].wait()
            pend_w[q] = pltpu.async_copy(
                rowsv[q], out_hbm.at[pl.ds(out_base_of[m], _CHUNK)], wsem[q])
    for m in range(max(0, _NTASK - _NBUF + 1), _NTASK):
        q = m % _NBUF
        pend_g[m].wait()
        pend_w[q] = pltpu.async_copy(
            rowsv[q], out_hbm.at[pl.ds(out_base_of[m], _CHUNK)], wsem[q])
    for q in range(_NBUF):
        if pend_w[q] is not None:
            pend_w[q].wait()


@functools.cache
def _build_sc_gather():
    mesh = plsc.VectorSubcoreMesh(core_axis_name="c", subcore_axis_name="s")
    return pl.kernel(
        _sc_gather_body,
        out_type=jax.ShapeDtypeStruct((NSLOT * B * S, H), jnp.float32),
        mesh=mesh,
        scratch_types=(
            [pltpu.VMEM((B * S + 16,), jnp.int32),
             pltpu.VMEM((_NTASK, _CHUNK), jnp.int32)]
            + [pltpu.VMEM((_CHUNK, H), jnp.float32) for _ in range(_NBUF)]
            + [pltpu.SemaphoreType.DMA for _ in range(2 * _NBUF)]
        ),
    )


_TBLK = 512


def _tc_body(f_ref, w_ref, b_ref, o_ref):
    tb = pl.program_id(1)
    acc = f_ref[6, 0] + b_ref[0][None, :]
    row = lax.broadcasted_iota(jnp.int32, (_TBLK, H), 0) + tb * _TBLK
    for k in range(6):
        n = k + 3
        f = f_ref[k, 0]
        f = jnp.where(row <= S - n, f, 0.0)
        wk = w_ref[:, k * H:(k + 1) * H]
        acc = acc + lax.dot_general(
            f, wk, (((1,), (1,)), ((), ())),
            preferred_element_type=jnp.float32)
    o_ref[0] = acc


_tc_project = pl.pallas_call(
    _tc_body,
    grid=(B, S // _TBLK),
    in_specs=[
        pl.BlockSpec((NSLOT, 1, _TBLK, H),
                     lambda b, t: (jnp.int32(0), b, t, jnp.int32(0))),
        pl.BlockSpec((H, 6 * H), lambda b, t: (jnp.int32(0), jnp.int32(0))),
        pl.BlockSpec((1, H), lambda b, t: (jnp.int32(0), jnp.int32(0))),
    ],
    out_specs=pl.BlockSpec((1, _TBLK, H), lambda b, t: (b, t, jnp.int32(0))),
    out_shape=jax.ShapeDtypeStruct((B, S, H), jnp.float32),
)


def kernel(x, byte_table, ngram_3, ngram_4, ngram_5, ngram_6, ngram_7,
           ngram_8, W, b):
    x32 = x.astype(jnp.int32).reshape(B * S)
    feats = _build_sc_gather()(x32, byte_table, ngram_3, ngram_4, ngram_5,
                               ngram_6, ngram_7, ngram_8)
    feats = feats.reshape(NSLOT, B, S, H)
    return _tc_project(feats, W, b.reshape(1, H))


# speedup vs baseline: 1.1077x; 1.0852x over previous
import functools

import jax
import jax.numpy as jnp
from jax import lax
from jax.experimental import pallas as pl
from jax.experimental.pallas import tpu as pltpu
from jax.experimental.pallas import tpu_sc as plsc

B = 4
S = 2048
H = 128
TAB = 500000
NSLOT = 7

_CMOD = (1, 256, 65536, 277216, 467296, 127776, 210656, 427936)
_WRAP = 448384

_NC = 2
_NS = 16
_NW = _NC * _NS

_CHUNK = 128
_CPB = S // _CHUNK
_TPW = (B * _CPB) // _NW
_NTASK = NSLOT * _TPW
_NBUF = 6


def _sc_gather_body(x_hbm, byte_hbm, t3, t4, t5, t6, t7, t8, out_hbm,
                    xv, idxv, rowsv0, rowsv1, rowsv2, rowsv3, rowsv4, rowsv5,
                    gsem0, gsem1, gsem2, gsem3, gsem4, gsem5,
                    wsem0, wsem1, wsem2, wsem3, wsem4, wsem5):
    wid = lax.axis_index("s") * _NC + lax.axis_index("c")
    pltpu.sync_copy(x_hbm, xv.at[pl.ds(0, B * S)])
    xv[pl.ds(B * S, 16)] = jnp.zeros((16,), jnp.int32)

    tables = (t3, t4, t5, t6, t7, t8, byte_hbm)
    c500k = jnp.full((16,), 500000, jnp.int32)
    zeros16 = jnp.zeros((16,), jnp.int32)
    wrap16 = jnp.full((16,), _WRAP, jnp.int32)

    rowsv = (rowsv0, rowsv1, rowsv2, rowsv3, rowsv4, rowsv5)
    gsem = (gsem0, gsem1, gsem2, gsem3, gsem4, gsem5)
    wsem = (wsem0, wsem1, wsem2, wsem3, wsem4, wsem5)

    tasks = [(slot, j) for slot in range(NSLOT) for j in range(_TPW)]

    pend_g = [None] * _NTASK
    pend_w = [None] * _NBUF
    out_base_of = [None] * _NTASK
    for i, (slot, j) in enumerate(tasks):
        p = i % _NBUF
        if pend_w[p] is not None:
            pend_w[p].wait()
            pend_w[p] = None
        n = slot + 3
        task = wid * _TPW + j
        b = task // _CPB
        t0 = (task - b * _CPB) * _CHUNK
        base = b * S + t0
        for g in range(_CHUNK // 16):
            off = base + g * 16
            if slot == 6:
                h = xv[pl.ds(off, 16)]
            else:
                acc = xv[pl.ds(off, 16)]
                for ii in range(1, n):
                    acc = acc + xv[pl.ds(off + ii, 16)] * _CMOD[ii]
                if n == 8:
                    x7 = xv[pl.ds(off + 7, 16)]
                    acc = acc + jnp.where(x7 >= 128, wrap16, zeros16)
                h = jnp.maximum(lax.rem(acc, c500k), zeros16)
            idxv[i, pl.ds(g * 16, 16)] = h
        out_base_of[i] = (slot * B + b) * S + t0
        pend_g[i] = pltpu.async_copy(
            tables[slot].at[idxv.at[jnp.int32(i)]], rowsv[p], gsem[p])
        m = i - (_NBUF - 1)
        if m >= 0:
            q = m % _NBUF
            pend_g[m].wait()
            pend_w[q] = pltpu.async_copy(
                rowsv[q], out_hbm.at[pl.ds(out_base_of[m], _CHUNK)], wsem[q])
    for m in range(max(0, _NTASK - _NBUF + 1), _NTASK):
        q = m % _NBUF
        pend_g[m].wait()
        pend_w[q] = pltpu.async_copy(
            rowsv[q], out_hbm.at[pl.ds(out_base_of[m], _CHUNK)], wsem[q])
    for q in range(_NBUF):
        if pend_w[q] is not None:
            pend_w[q].wait()


@functools.cache
def _build_sc_gather():
    mesh = plsc.VectorSubcoreMesh(core_axis_name="c", subcore_axis_name="s")
    return pl.kernel(
        _sc_gather_body,
        out_type=jax.ShapeDtypeStruct((NSLOT * B * S, H), jnp.float32),
        mesh=mesh,
        scratch_types=(
            [pltpu.VMEM((B * S + 16,), jnp.int32),
             pltpu.VMEM((_NTASK, _CHUNK), jnp.int32)]
            + [pltpu.VMEM((_CHUNK, H), jnp.float32) for _ in range(_NBUF)]
            + [pltpu.SemaphoreType.DMA for _ in range(2 * _NBUF)]
        ),
    )


_TBLK = 512


def _tc_body(f_ref, w_ref, b_ref, o_ref):
    tb = pl.program_id(1)
    acc = f_ref[6, 0] + b_ref[0][None, :]
    row = lax.broadcasted_iota(jnp.int32, (_TBLK, H), 0) + tb * _TBLK
    for k in range(6):
        n = k + 3
        f = f_ref[k, 0]
        f = jnp.where(row <= S - n, f, 0.0)
        wk = w_ref[:, k * H:(k + 1) * H]
        acc = acc + lax.dot_general(
            f, wk, (((1,), (1,)), ((), ())),
            preferred_element_type=jnp.float32)
    o_ref[0] = acc


_tc_project = pl.pallas_call(
    _tc_body,
    grid=(B, S // _TBLK),
    in_specs=[
        pl.BlockSpec((NSLOT, 1, _TBLK, H),
                     lambda b, t: (jnp.int32(0), b, t, jnp.int32(0))),
        pl.BlockSpec((H, 6 * H), lambda b, t: (jnp.int32(0), jnp.int32(0))),
        pl.BlockSpec((1, H), lambda b, t: (jnp.int32(0), jnp.int32(0))),
    ],
    out_specs=pl.BlockSpec((1, _TBLK, H), lambda b, t: (b, t, jnp.int32(0))),
    out_shape=jax.ShapeDtypeStruct((B, S, H), jnp.float32),
)


def kernel(x, byte_table, ngram_3, ngram_4, ngram_5, ngram_6, ngram_7,
           ngram_8, W, b):
    x32 = x.astype(jnp.int32).reshape(B * S)
    feats = _build_sc_gather()(x32, byte_table, ngram_3, ngram_4, ngram_5,
                               ngram_6, ngram_7, ngram_8)
    feats = feats.reshape(NSLOT, B, S, H)
    return _tc_project(feats, W, b.reshape(1, H))
